# Initial kernel scaffold; baseline (speedup 1.0000x reference)
import jax
import jax.numpy as jnp
from jax.experimental import pallas as pl


def kernel(x, table):
    raise NotImplementedError("write your pallas kernel here")


# speedup vs baseline: 2.9768x; 2.9768x over previous
import functools

import jax
import jax.numpy as jnp
from jax import lax
from jax.experimental import pallas as pl
from jax.experimental.pallas import tpu as pltpu
from jax.experimental.pallas import tpu_sc as plsc

EMB = 128
NC = 2
NS = 16
NW = NC * NS
CH = 128


def _emb_kernel_body(n_chunks, per_w, x_hbm, tab_hbm, out_hbm, idx_v, buf, gsem):
    wid = lax.axis_index("s") * NC + lax.axis_index("c")
    pltpu.sync_copy(x_hbm.at[wid], idx_v)
    base = wid * per_w

    @pl.loop(0, n_chunks)
    def _chunk(j):
        pltpu.async_copy(tab_hbm.at[idx_v.at[j]], buf, gsem).wait()
        pltpu.sync_copy(buf, out_hbm.at[pl.ds(base + j * CH, CH)])


def _make_emb_call(tot):
    per_w = tot // NW
    n_chunks = per_w // CH
    mesh = plsc.VectorSubcoreMesh(core_axis_name="c", subcore_axis_name="s")
    return pl.kernel(
        functools.partial(_emb_kernel_body, n_chunks, per_w),
        out_type=jax.ShapeDtypeStruct((tot, EMB), jnp.float32),
        mesh=mesh,
        scratch_types=[
            pltpu.VMEM((n_chunks, CH), jnp.int32),
            pltpu.VMEM((CH, EMB), jnp.float32),
            pltpu.SemaphoreType.DMA,
        ],
    )


def kernel(x, table):
    b, h = x.shape
    tot = b * h
    xf = x.reshape(NW, tot // (NW * CH), CH).astype(jnp.int32)
    out = _make_emb_call(tot)(xf, table)
    return out.reshape(b, h, EMB)


# speedup vs baseline: 3.3719x; 1.1327x over previous
import functools

import jax
import jax.numpy as jnp
from jax import lax
from jax.experimental import pallas as pl
from jax.experimental.pallas import tpu as pltpu
from jax.experimental.pallas import tpu_sc as plsc

EMB = 128
NC = 2
NS = 16
NW = NC * NS
CH = 128
NBUF = 5
DEPTH = 3


def _emb_kernel_body(n_chunks, per_w, x_hbm, tab_hbm, out_hbm, idx_v, bufs,
                     gsem, osem):
    wid = lax.axis_index("s") * NC + lax.axis_index("c")
    pltpu.sync_copy(x_hbm.at[wid], idx_v)
    base = wid * per_w

    def gather(c, slot):
        pltpu.async_copy(tab_hbm.at[idx_v.at[c]], bufs.at[slot], gsem)

    def out_slice(c):
        return out_hbm.at[pl.ds(base + c * CH, CH)]

    for c in range(DEPTH):
        gather(c, c)

    @pl.loop(0, n_chunks, step=NBUF)
    def _group(g):
        for b in range(NBUF):
            j = g + b

            @pl.when(j >= NBUF - DEPTH)
            def _():
                pltpu.make_async_copy(bufs.at[b], out_slice(j), osem).wait()

            pltpu.make_async_copy(tab_hbm.at[idx_v.at[j]], bufs.at[b],
                                  gsem).wait()
            pltpu.async_copy(bufs.at[b], out_slice(j), osem)

            @pl.when(j + DEPTH < n_chunks)
            def _():
                gather(j + DEPTH, (b + DEPTH) % NBUF)

    for b in range(NBUF - DEPTH):
        pltpu.make_async_copy(bufs.at[b], out_slice(0), osem).wait()


def _make_emb_call(tot):
    per_w = tot // NW
    n_chunks = per_w // CH
    mesh = plsc.VectorSubcoreMesh(core_axis_name="c", subcore_axis_name="s")
    return pl.kernel(
        functools.partial(_emb_kernel_body, n_chunks, per_w),
        out_type=jax.ShapeDtypeStruct((tot, EMB), jnp.float32),
        mesh=mesh,
        scratch_types=[
            pltpu.VMEM((n_chunks, CH), jnp.int32),
            pltpu.VMEM((NBUF, CH, EMB), jnp.float32),
            pltpu.SemaphoreType.DMA,
            pltpu.SemaphoreType.DMA,
        ],
    )


def kernel(x, table):
    b, h = x.shape
    tot = b * h
    xf = x.reshape(NW, tot // (NW * CH), CH).astype(jnp.int32)
    out = _make_emb_call(tot)(xf, table)
    return out.reshape(b, h, EMB)


# speedup vs baseline: 6.0283x; 1.7878x over previous
import functools

import jax
import jax.numpy as jnp
from jax import lax
from jax.experimental import pallas as pl
from jax.experimental.pallas import tpu as pltpu
from jax.experimental.pallas import tpu_sc as plsc

EMB = 128
NC = 2
NS = 16
NW = NC * NS
NBUF = 8
DEPTH = 6


def _emb_kernel_body(n_chunks, rows_per_w, h, x_hbm, tab_hbm, out_hbm, idx_v,
                     bufs, gsem, osem):
    wid = lax.axis_index("s") * NC + lax.axis_index("c")
    base = wid * rows_per_w
    pltpu.sync_copy(x_hbm.at[pl.ds(base, rows_per_w)], idx_v)

    def gather(c, slot):
        pltpu.async_copy(tab_hbm.at[idx_v.at[c]], bufs.at[slot], gsem)

    def out_slice(c):
        return out_hbm.at[base + c]

    for c in range(DEPTH):
        gather(c, c)

    @pl.loop(0, n_chunks, step=NBUF)
    def _group(g):
        for b in range(NBUF):
            j = g + b

            @pl.when(j >= NBUF - DEPTH)
            def _():
                pltpu.make_async_copy(bufs.at[b], out_slice(j), osem).wait()

            pltpu.make_async_copy(tab_hbm.at[idx_v.at[j]], bufs.at[b],
                                  gsem).wait()
            pltpu.async_copy(bufs.at[b], out_slice(j), osem)

            @pl.when(j + DEPTH < n_chunks)
            def _():
                gather(j + DEPTH, (b + DEPTH) % NBUF)

    for b in range(NBUF - DEPTH):
        pltpu.make_async_copy(bufs.at[b], out_slice(0), osem).wait()


def _make_emb_call(bsz, h):
    rows_per_w = bsz // NW
    n_chunks = rows_per_w
    mesh = plsc.VectorSubcoreMesh(core_axis_name="c", subcore_axis_name="s")
    return pl.kernel(
        functools.partial(_emb_kernel_body, n_chunks, rows_per_w, h),
        out_type=jax.ShapeDtypeStruct((bsz, h, EMB), jnp.float32),
        mesh=mesh,
        scratch_types=[
            pltpu.VMEM((rows_per_w, h), jnp.int32),
            pltpu.VMEM((NBUF, h, EMB), jnp.float32),
            pltpu.SemaphoreType.DMA,
            pltpu.SemaphoreType.DMA,
        ],
    )


def kernel(x, table):
    b, h = x.shape
    return _make_emb_call(b, h)(x.astype(jnp.int32), table)


# speedup vs baseline: 6.0291x; 1.0001x over previous
import functools

import jax
import jax.numpy as jnp
from jax import lax
from jax.experimental import pallas as pl
from jax.experimental.pallas import tpu as pltpu
from jax.experimental.pallas import tpu_sc as plsc

EMB = 128
NC = 2
NS = 16
NW = NC * NS
NBUF = 8
DEPTH = 6


def _emb_kernel_body(n_chunks, rows_per_w, h, x_hbm, tab_hbm, out_hbm, idx_v,
                     bufs, gsem, osem):
    wid = lax.axis_index("s") * NC + lax.axis_index("c")
    base = wid * rows_per_w
    pltpu.sync_copy(x_hbm.at[pl.ds(base, rows_per_w)], idx_v)

    def gather(c, slot):
        pltpu.async_copy(tab_hbm.at[idx_v.at[c]], bufs.at[slot], gsem)

    def out_slice(c):
        return out_hbm.at[base + c]

    for c in range(DEPTH):
        gather(c, c)

    @pl.loop(0, n_chunks, step=NBUF)
    def _group(g):
        for b in range(NBUF):
            j = g + b

            @pl.when(j >= NBUF - DEPTH)
            def _():
                pltpu.make_async_copy(bufs.at[b], out_slice(j), osem).wait()

            pltpu.make_async_copy(tab_hbm.at[idx_v.at[j]], bufs.at[b],
                                  gsem).wait()
            pltpu.async_copy(bufs.at[b], out_slice(j), osem)

            @pl.when(j + DEPTH < n_chunks)
            def _():
                gather(j + DEPTH, (b + DEPTH) % NBUF)

    for b in range(NBUF - DEPTH):
        pltpu.make_async_copy(bufs.at[b], out_slice(0), osem).wait()


def _make_emb_call(bsz, h):
    rows_per_w = bsz // NW
    n_chunks = rows_per_w
    mesh = plsc.VectorSubcoreMesh(core_axis_name="c", subcore_axis_name="s")
    return pl.kernel(
        functools.partial(_emb_kernel_body, n_chunks, rows_per_w, h),
        out_type=jax.ShapeDtypeStruct((bsz, h, EMB), jnp.float32),
        mesh=mesh,
        compiler_params=pltpu.CompilerParams(use_tc_tiling_on_sc=True),
        scratch_types=[
            pltpu.VMEM((rows_per_w, h), jnp.int32),
            pltpu.VMEM((NBUF, h, EMB), jnp.float32),
            pltpu.SemaphoreType.DMA,
            pltpu.SemaphoreType.DMA,
        ],
    )


def kernel(x, table):
    b, h = x.shape
    return _make_emb_call(b, h)(x.astype(jnp.int32), table)
